# baseline (speedup 1.0000x reference)
import functools

import jax
import jax.numpy as jnp
from jax import lax
from jax.experimental import pallas as pl
from jax.experimental.pallas import tpu as pltpu
from jax.experimental.pallas import tpu_sc as plsc

NC = 2
NS = 16
NW = NC * NS
CH = 96
BLK = 6


def _make_deg_kernel(npad, k_chunks):
    rpt = npad // NS
    mesh = plsc.VectorSubcoreMesh(core_axis_name="c", subcore_axis_name="s")

    @functools.partial(
        pl.kernel,
        mesh=mesh,
        out_type=jax.ShapeDtypeStruct((NC, npad, 16), jnp.float32),
        scratch_types=[
            pltpu.VMEM((k_chunks // BLK, BLK, CH), jnp.int32),
            pltpu.VMEM((CH, 16), jnp.float32),
            pltpu.VMEM((CH, 16), jnp.float32),
            pltpu.VMEM_SHARED((npad, 16), jnp.float32),
        ],
    )
    def deg_kernel(dst_hbm, out_hbm, idx_d, ones_v, zero_v, acc):
        c = lax.axis_index("c")
        s = lax.axis_index("s")
        wid = s * NC + c

        def init_rows(i, _):
            ones_v[i] = jnp.ones((16,), jnp.float32)
            zero_v[i] = jnp.zeros((16,), jnp.float32)
            return 0

        lax.fori_loop(0, CH, init_rows, 0)
        r0 = s * rpt

        def zero_acc(j, _):
            pltpu.sync_copy(zero_v, acc.at[pl.ds(r0 + j * CH, CH)])
            return 0

        lax.fori_loop(0, rpt // CH, zero_acc, 0)
        plsc.subcore_barrier()

        pltpu.sync_copy(dst_hbm.at[wid], idx_d)

        def body(jb, _):
            for t in range(BLK):
                pltpu.sync_copy(ones_v, acc.at[idx_d.at[jb, t]], add=True)
            return 0

        lax.fori_loop(0, k_chunks // BLK, body, 0)
        plsc.subcore_barrier()
        pltpu.sync_copy(acc.at[pl.ds(r0, rpt)], out_hbm.at[c, pl.ds(r0, rpt)])

    return deg_kernel


def _make_scatter_kernel(npad, k_chunks, h):
    rpt = npad // NS
    mesh = plsc.VectorSubcoreMesh(core_axis_name="c", subcore_axis_name="s")

    @functools.partial(
        pl.kernel,
        mesh=mesh,
        out_type=jax.ShapeDtypeStruct((NC, npad, h), jnp.float32),
        scratch_types=[
            pltpu.VMEM((2, BLK, CH), jnp.int32),
            pltpu.VMEM((2, BLK, CH), jnp.int32),
            pltpu.VMEM((CH, h), jnp.float32),
            pltpu.VMEM((CH, h), jnp.float32),
            pltpu.VMEM((CH, h), jnp.float32),
            pltpu.VMEM_SHARED((npad, h), jnp.float32),
            pltpu.SemaphoreType.DMA,
            pltpu.SemaphoreType.DMA,
            pltpu.SemaphoreType.DMA,
            pltpu.SemaphoreType.DMA,
            pltpu.SemaphoreType.DMA,
            pltpu.SemaphoreType.DMA,
            pltpu.SemaphoreType.DMA,
        ],
    )
    def scatter_kernel(hs_hbm, src_hbm, dst_hbm, out_hbm,
                       is2, id2, r0b, r1b, r2b, acc,
                       sg0, sg1, sg2, ss0, ss1, ss2, si):
        rows = (r0b, r1b, r2b)
        sg = (sg0, sg1, sg2)
        ss = (ss0, ss1, ss2)
        nb = k_chunks // BLK
        c = lax.axis_index("c")
        s = lax.axis_index("s")
        wid = s * NC + c

        def zr(i, _):
            def zc(j, _2):
                r0b[i, pl.ds(j * 16, 16)] = jnp.zeros((16,), jnp.float32)
                return 0

            return lax.fori_loop(0, h // 16, zc, 0)

        lax.fori_loop(0, CH, zr, 0)
        r0 = s * rpt
        nfull = rpt // CH
        rem = rpt - nfull * CH

        def zero_acc(j, _):
            pltpu.sync_copy(r0b, acc.at[pl.ds(r0 + j * CH, CH)])
            return 0

        lax.fori_loop(0, nfull, zero_acc, 0)
        if rem:
            pltpu.sync_copy(r0b.at[pl.ds(0, rem)],
                            acc.at[pl.ds(r0 + nfull * CH, rem)])
        plsc.subcore_barrier()

        pltpu.sync_copy(src_hbm.at[wid, 0], is2.at[0])
        pltpu.sync_copy(dst_hbm.at[wid, 0], id2.at[0])
        pltpu.async_copy(hs_hbm.at[is2.at[0, 0]], rows[0], sg[0])
        pltpu.async_copy(hs_hbm.at[is2.at[0, 1]], rows[1], sg[1])

        def process_block(bidx, slab, first_pred, tail_pred):
            oslab = 1 - slab
            for t in range(BLK):
                slot = t % 3
                nslot = (t + 2) % 3
                pltpu.make_async_copy(
                    hs_hbm.at[is2.at[slab, t]], rows[slot], sg[slot]).wait()
                pltpu.async_copy(rows[slot], acc.at[id2.at[slab, t]],
                                 ss[slot], add=True)
                if t == 0:
                    def wait_prev():
                        pltpu.make_async_copy(
                            rows[nslot], acc.at[id2.at[slab, t]],
                            ss[nslot]).wait()

                    def prefetch():
                        pltpu.async_copy(src_hbm.at[wid, bidx + 1],
                                         is2.at[oslab], si)
                        pltpu.async_copy(dst_hbm.at[wid, bidx + 1],
                                         id2.at[oslab], si)

                    if first_pred is None:
                        wait_prev()
                    else:
                        pl.when(first_pred)(wait_prev)
                    if tail_pred is None:
                        prefetch()
                    else:
                        pl.when(tail_pred)(prefetch)
                    pltpu.async_copy(hs_hbm.at[is2.at[slab, t + 2]],
                                     rows[nslot], sg[nslot])
                elif t < 4:
                    pltpu.make_async_copy(
                        rows[nslot], acc.at[id2.at[slab, t]],
                        ss[nslot]).wait()
                    pltpu.async_copy(hs_hbm.at[is2.at[slab, t + 2]],
                                     rows[nslot], sg[nslot])
                else:
                    def tail():
                        if t == 4:
                            pltpu.make_async_copy(
                                src_hbm.at[wid, 0], is2.at[oslab], si).wait()
                            pltpu.make_async_copy(
                                dst_hbm.at[wid, 0], id2.at[oslab], si).wait()
                        pltpu.make_async_copy(
                            rows[nslot], acc.at[id2.at[slab, t]],
                            ss[nslot]).wait()
                        pltpu.async_copy(hs_hbm.at[is2.at[oslab, t - 4]],
                                         rows[nslot], sg[nslot])

                    if tail_pred is None:
                        tail()
                    else:
                        pl.when(tail_pred)(tail)

        nb2 = nb // 2

        def body(j2, _):
            b0 = 2 * j2
            process_block(b0, 0, first_pred=j2 > 0, tail_pred=None)
            process_block(b0 + 1, 1, first_pred=None,
                          tail_pred=j2 < nb2 - 1)
            return 0

        lax.fori_loop(0, nb2, body, 0)
        for slot in range(3):
            pltpu.make_async_copy(rows[slot], acc.at[id2.at[0, 0]],
                                  ss[slot]).wait()
        plsc.subcore_barrier()
        pltpu.sync_copy(acc.at[pl.ds(r0, rpt)], out_hbm.at[c, pl.ds(r0, rpt)])

    return scatter_kernel


def _dinv_from_parts(degp, n):
    deg = degp[0, :n, 0:1] + degp[1, :n, 0:1] + 1.0
    return lax.rsqrt(deg)


def _tc_input_body(n, npad, x_ref, iw_ref, ib_ref, cw0_ref, degp_ref,
                   xc_ref, hs0_ref):
    x = x_ref[...]
    h = jnp.dot(x, iw_ref[...], preferred_element_type=jnp.float32) + ib_ref[...]
    xc_ref[...] = h
    dinv = _dinv_from_parts(degp_ref[...], n)
    hs0 = dinv * jnp.dot(h, cw0_ref[...], preferred_element_type=jnp.float32)
    hs0_ref[...] = jnp.concatenate(
        [hs0, jnp.zeros((npad - n, hs0.shape[1]), jnp.float32)], axis=0)


def _tc_layer_body(n, npad, has_prev, refs):
    if has_prev:
        (part_ref, hs_ref, degp_ref, xc_ref, prev_ref, cb_ref, g_ref, b_ref,
         cwn_ref, cur_ref, hsn_ref) = refs
    else:
        (part_ref, hs_ref, degp_ref, xc_ref, cb_ref, g_ref, b_ref,
         cwn_ref, cur_ref, hsn_ref) = refs
    part = part_ref[...]
    hs = hs_ref[...]
    dinv = _dinv_from_parts(degp_ref[...], n)
    agg = dinv * (part[0, :n] + part[1, :n] + hs[:n]) + cb_ref[...]
    mu = jnp.mean(agg, axis=0, keepdims=True)
    ce = agg - mu
    var = jnp.mean(ce * ce, axis=0, keepdims=True)
    bn = g_ref[...] * ce * lax.rsqrt(var + 1e-5) + b_ref[...]
    r = jnp.maximum(bn, 0.0)
    cur = r + 0.2 * xc_ref[...]
    if has_prev:
        cur = cur + 0.5 * prev_ref[...]
    cur_ref[...] = cur
    hsn = dinv * jnp.dot(cur, cwn_ref[...], preferred_element_type=jnp.float32)
    hsn_ref[...] = jnp.concatenate(
        [hsn, jnp.zeros((npad - n, hsn.shape[1]), jnp.float32)], axis=0)


def _tc_final_body(n, part_ref, hs_ref, degp_ref, xc_ref, l0_ref, l1_ref,
                   cb_ref, g_ref, b_ref, lw_ref, ow_ref, ob_ref, out_ref):
    part = part_ref[...]
    hs = hs_ref[...]
    dinv = _dinv_from_parts(degp_ref[...], n)
    agg = dinv * (part[0, :n] + part[1, :n] + hs[:n]) + cb_ref[...]
    mu = jnp.mean(agg, axis=0, keepdims=True)
    ce = agg - mu
    var = jnp.mean(ce * ce, axis=0, keepdims=True)
    bn = g_ref[...] * ce * lax.rsqrt(var + 1e-5) + b_ref[...]
    r = jnp.maximum(bn, 0.0)
    cur2 = r + 0.2 * xc_ref[...] + 0.5 * l1_ref[...]

    lw = lw_ref[...]
    m = jnp.max(lw, axis=-1, keepdims=True)
    e = jnp.exp(lw - m)
    w = e / jnp.sum(e, axis=-1, keepdims=True)
    comb = (w[0:1, 0:1] * l0_ref[...] + w[0:1, 1:2] * l1_ref[...]
            + w[0:1, 2:3] * cur2)

    logits = jnp.dot(comb, ow_ref[...],
                     preferred_element_type=jnp.float32) + ob_ref[...]
    mx = jnp.max(logits, axis=-1, keepdims=True)
    sh = logits - mx
    lse = jnp.log(jnp.sum(jnp.exp(sh), axis=-1, keepdims=True))
    out_ref[...] = sh - lse


def kernel(x, adj_m, input_W, input_b, conv_W, conv_b, bn_gamma, bn_beta,
           output_W, output_b, layer_weights):
    n, d_in = x.shape
    h = input_W.shape[1]
    e = adj_m.shape[1]
    out_dim = output_W.shape[1]
    nl = conv_W.shape[0]

    npad = ((n + 1 + NS * 128 - 1) // (NS * 128)) * (NS * 128)
    kc = (e + NW * CH - 1) // (NW * CH)
    k_chunks = ((kc + 2 * BLK - 1) // (2 * BLK)) * (2 * BLK)
    ep = k_chunks * NW * CH

    src = adj_m[0]
    dst = adj_m[1]
    pad = jnp.full((ep - e,), n, dtype=jnp.int32)
    srcp = jnp.concatenate([src, pad]).reshape(NW, k_chunks // BLK, BLK, CH)
    dstp = jnp.concatenate([dst, pad]).reshape(NW, k_chunks // BLK, BLK, CH)

    deg_k = _make_deg_kernel(npad, k_chunks)
    scat_k = _make_scatter_kernel(npad, k_chunks, h)

    degp = deg_k(dstp)

    ib = input_b.reshape(1, h)
    xc, hs = pl.pallas_call(
        functools.partial(_tc_input_body, n, npad),
        out_shape=[
            jax.ShapeDtypeStruct((n, h), jnp.float32),
            jax.ShapeDtypeStruct((npad, h), jnp.float32),
        ],
    )(x, input_W, ib, conv_W[0], degp)

    lst = []
    for i in range(nl - 1):
        part = scat_k(hs, srcp, dstp)
        body = functools.partial(_tc_layer_body, n, npad, i > 0)
        args = [part, hs, degp, xc]
        if i > 0:
            args.append(lst[-1])
        args += [conv_b[i].reshape(1, h), bn_gamma[i].reshape(1, h),
                 bn_beta[i].reshape(1, h), conv_W[i + 1]]
        cur, hs = pl.pallas_call(
            lambda *refs, _b=body: _b(refs),
            out_shape=[
                jax.ShapeDtypeStruct((n, h), jnp.float32),
                jax.ShapeDtypeStruct((npad, h), jnp.float32),
            ],
        )(*args)
        lst.append(cur)

    part = scat_k(hs, srcp, dstp)
    lw = jnp.concatenate(
        [layer_weights.reshape(1, nl),
         jnp.full((1, h - nl), -1e30, jnp.float32)], axis=1)
    i = nl - 1
    out = pl.pallas_call(
        functools.partial(_tc_final_body, n),
        out_shape=jax.ShapeDtypeStruct((n, out_dim), jnp.float32),
    )(part, hs, degp, xc, lst[0], lst[1], conv_b[i].reshape(1, h),
      bn_gamma[i].reshape(1, h), bn_beta[i].reshape(1, h), lw,
      output_W, output_b.reshape(1, out_dim))
    return out


# speedup vs baseline: 2.6158x; 2.6158x over previous
import functools

import jax
import jax.numpy as jnp
from jax import lax
from jax.experimental import pallas as pl
from jax.experimental.pallas import tpu as pltpu
from jax.experimental.pallas import tpu_sc as plsc

NC = 2
NS = 16
NW = NC * NS
CH = 120
BLK = 6

NB0 = 22
NB1 = 6


def _make_deg_kernel(npad):
    rpt = npad // NS
    mesh = plsc.VectorSubcoreMesh(core_axis_name="c", subcore_axis_name="s")

    @functools.partial(
        pl.kernel,
        mesh=mesh,
        out_type=jax.ShapeDtypeStruct((NC, npad, 16), jnp.float32),
        scratch_types=[
            pltpu.VMEM((1, BLK, CH), jnp.int32),
            pltpu.VMEM((CH, 16), jnp.float32),
            pltpu.VMEM((CH, 16), jnp.float32),
            pltpu.VMEM_SHARED((npad, 16), jnp.float32),
        ],
    )
    def deg_kernel(dst_hbm, out_hbm, idx_d, ones_v, zero_v, acc):
        c = lax.axis_index("c")
        s = lax.axis_index("s")

        def init_rows(i, _):
            ones_v[i] = jnp.ones((16,), jnp.float32)
            zero_v[i] = jnp.zeros((16,), jnp.float32)
            return 0

        lax.fori_loop(0, CH, init_rows, 0)
        r0 = s * rpt
        nfull = rpt // CH
        rem = rpt - nfull * CH

        def zero_acc(j, _):
            pltpu.sync_copy(zero_v, acc.at[pl.ds(r0 + j * CH, CH)])
            return 0

        lax.fori_loop(0, nfull, zero_acc, 0)
        if rem:
            pltpu.sync_copy(zero_v.at[pl.ds(0, rem)],
                            acc.at[pl.ds(r0 + nfull * CH, rem)])
        plsc.subcore_barrier()

        def run(base, nb_static):
            def body(jb, _):
                pltpu.sync_copy(dst_hbm.at[base + jb], idx_d.at[0])
                for t in range(BLK):
                    pltpu.sync_copy(ones_v, acc.at[idx_d.at[0, t]], add=True)
                return 0

            lax.fori_loop(0, nb_static, body, 0)

        pl.when(c == 0)(lambda: run(s * NB0, NB0))
        pl.when(c == 1)(lambda: run(NS * NB0 + s * NB1, NB1))
        plsc.subcore_barrier()
        pltpu.sync_copy(acc.at[pl.ds(r0, rpt)], out_hbm.at[c, pl.ds(r0, rpt)])

    return deg_kernel


def _make_scatter_kernel(npad, h):
    rpt = npad // NS
    mesh = plsc.VectorSubcoreMesh(core_axis_name="c", subcore_axis_name="s")

    @functools.partial(
        pl.kernel,
        mesh=mesh,
        out_type=jax.ShapeDtypeStruct((NC, npad, h), jnp.float32),
        scratch_types=[
            pltpu.VMEM((2, BLK, CH), jnp.int32),
            pltpu.VMEM((2, BLK, CH), jnp.int32),
            pltpu.VMEM((CH, h), jnp.float32),
            pltpu.VMEM((CH, h), jnp.float32),
            pltpu.VMEM((CH, h), jnp.float32),
            pltpu.VMEM_SHARED((npad, h), jnp.float32),
            pltpu.SemaphoreType.DMA,
            pltpu.SemaphoreType.DMA,
            pltpu.SemaphoreType.DMA,
            pltpu.SemaphoreType.DMA,
            pltpu.SemaphoreType.DMA,
            pltpu.SemaphoreType.DMA,
            pltpu.SemaphoreType.DMA,
        ],
    )
    def scatter_kernel(hs_hbm, src_hbm, dst_hbm, out_hbm,
                       is2, id2, r0b, r1b, r2b, acc,
                       sg0, sg1, sg2, ss0, ss1, ss2, si):
        rows = (r0b, r1b, r2b)
        sg = (sg0, sg1, sg2)
        ss = (ss0, ss1, ss2)
        c = lax.axis_index("c")
        s = lax.axis_index("s")

        def zr(i, _):
            def zc(j, _2):
                r0b[i, pl.ds(j * 16, 16)] = jnp.zeros((16,), jnp.float32)
                return 0

            return lax.fori_loop(0, h // 16, zc, 0)

        lax.fori_loop(0, CH, zr, 0)
        r0 = s * rpt
        nfull = rpt // CH
        rem = rpt - nfull * CH

        def zero_acc(j, _):
            pltpu.sync_copy(r0b, acc.at[pl.ds(r0 + j * CH, CH)])
            return 0

        lax.fori_loop(0, nfull, zero_acc, 0)
        if rem:
            pltpu.sync_copy(r0b.at[pl.ds(0, rem)],
                            acc.at[pl.ds(r0 + nfull * CH, rem)])
        plsc.subcore_barrier()

        def process_block(bidx, slab, first_pred, tail_pred):
            oslab = 1 - slab
            for t in range(BLK):
                slot = t % 3
                nslot = (t + 2) % 3
                pltpu.make_async_copy(
                    hs_hbm.at[is2.at[slab, t]], rows[slot], sg[slot]).wait()
                pltpu.async_copy(rows[slot], acc.at[id2.at[slab, t]],
                                 ss[slot], add=True)
                if t == 0:
                    def wait_prev():
                        pltpu.make_async_copy(
                            rows[nslot], acc.at[id2.at[slab, t]],
                            ss[nslot]).wait()

                    def prefetch():
                        pltpu.async_copy(src_hbm.at[bidx + 1],
                                         is2.at[oslab], si)
                        pltpu.async_copy(dst_hbm.at[bidx + 1],
                                         id2.at[oslab], si)

                    if first_pred is None:
                        wait_prev()
                    else:
                        pl.when(first_pred)(wait_prev)
                    if tail_pred is None:
                        prefetch()
                    else:
                        pl.when(tail_pred)(prefetch)
                    pltpu.async_copy(hs_hbm.at[is2.at[slab, t + 2]],
                                     rows[nslot], sg[nslot])
                elif t < 4:
                    pltpu.make_async_copy(
                        rows[nslot], acc.at[id2.at[slab, t]],
                        ss[nslot]).wait()
                    pltpu.async_copy(hs_hbm.at[is2.at[slab, t + 2]],
                                     rows[nslot], sg[nslot])
                else:
                    def tail():
                        if t == 4:
                            pltpu.make_async_copy(
                                src_hbm.at[0], is2.at[oslab], si).wait()
                            pltpu.make_async_copy(
                                dst_hbm.at[0], id2.at[oslab], si).wait()
                        pltpu.make_async_copy(
                            rows[nslot], acc.at[id2.at[slab, t]],
                            ss[nslot]).wait()
                        pltpu.async_copy(hs_hbm.at[is2.at[oslab, t - 4]],
                                         rows[nslot], sg[nslot])

                    if tail_pred is None:
                        tail()
                    else:
                        pl.when(tail_pred)(tail)

        def pipeline(base, nb_static):
            nb2s = nb_static // 2
            pltpu.sync_copy(src_hbm.at[base], is2.at[0])
            pltpu.sync_copy(dst_hbm.at[base], id2.at[0])
            pltpu.async_copy(hs_hbm.at[is2.at[0, 0]], rows[0], sg[0])
            pltpu.async_copy(hs_hbm.at[is2.at[0, 1]], rows[1], sg[1])

            def body(j2, _):
                b0 = base + 2 * j2
                process_block(b0, 0, first_pred=j2 > 0, tail_pred=None)
                process_block(b0 + 1, 1, first_pred=None,
                              tail_pred=j2 < nb2s - 1)
                return 0

            lax.fori_loop(0, nb2s, body, 0)
            for slot in range(3):
                pltpu.make_async_copy(rows[slot], acc.at[id2.at[0, 0]],
                                      ss[slot]).wait()

        pl.when(c == 0)(lambda: pipeline(s * NB0, NB0))
        pl.when(c == 1)(lambda: pipeline(NS * NB0 + s * NB1, NB1))
        plsc.subcore_barrier()
        pltpu.sync_copy(acc.at[pl.ds(r0, rpt)], out_hbm.at[c, pl.ds(r0, rpt)])

    return scatter_kernel


def _dinv_from_parts(degp, n):
    deg = degp[0, :n, 0:1] + degp[1, :n, 0:1] + 1.0
    return lax.rsqrt(deg)


def _tc_input_body(n, npad, x_ref, iw_ref, ib_ref, cw0_ref, degp_ref,
                   xc_ref, hs0_ref):
    x = x_ref[...]
    h = jnp.dot(x, iw_ref[...], preferred_element_type=jnp.float32) + ib_ref[...]
    xc_ref[...] = h
    dinv = _dinv_from_parts(degp_ref[...], n)
    hs0 = dinv * jnp.dot(h, cw0_ref[...], preferred_element_type=jnp.float32)
    hs0_ref[...] = jnp.concatenate(
        [hs0, jnp.zeros((npad - n, hs0.shape[1]), jnp.float32)], axis=0)


def _tc_layer_body(n, npad, has_prev, refs):
    if has_prev:
        (part_ref, hs_ref, degp_ref, xc_ref, prev_ref, cb_ref, g_ref, b_ref,
         cwn_ref, cur_ref, hsn_ref) = refs
    else:
        (part_ref, hs_ref, degp_ref, xc_ref, cb_ref, g_ref, b_ref,
         cwn_ref, cur_ref, hsn_ref) = refs
    part = part_ref[...]
    hs = hs_ref[...]
    dinv = _dinv_from_parts(degp_ref[...], n)
    agg = dinv * (part[0, :n] + part[1, :n] + hs[:n]) + cb_ref[...]
    mu = jnp.mean(agg, axis=0, keepdims=True)
    ce = agg - mu
    var = jnp.mean(ce * ce, axis=0, keepdims=True)
    bn = g_ref[...] * ce * lax.rsqrt(var + 1e-5) + b_ref[...]
    r = jnp.maximum(bn, 0.0)
    cur = r + 0.2 * xc_ref[...]
    if has_prev:
        cur = cur + 0.5 * prev_ref[...]
    cur_ref[...] = cur
    hsn = dinv * jnp.dot(cur, cwn_ref[...], preferred_element_type=jnp.float32)
    hsn_ref[...] = jnp.concatenate(
        [hsn, jnp.zeros((npad - n, hsn.shape[1]), jnp.float32)], axis=0)


def _tc_final_body(n, part_ref, hs_ref, degp_ref, xc_ref, l0_ref, l1_ref,
                   cb_ref, g_ref, b_ref, lw_ref, ow_ref, ob_ref, out_ref):
    part = part_ref[...]
    hs = hs_ref[...]
    dinv = _dinv_from_parts(degp_ref[...], n)
    agg = dinv * (part[0, :n] + part[1, :n] + hs[:n]) + cb_ref[...]
    mu = jnp.mean(agg, axis=0, keepdims=True)
    ce = agg - mu
    var = jnp.mean(ce * ce, axis=0, keepdims=True)
    bn = g_ref[...] * ce * lax.rsqrt(var + 1e-5) + b_ref[...]
    r = jnp.maximum(bn, 0.0)
    cur2 = r + 0.2 * xc_ref[...] + 0.5 * l1_ref[...]

    lw = lw_ref[...]
    m = jnp.max(lw, axis=-1, keepdims=True)
    e = jnp.exp(lw - m)
    w = e / jnp.sum(e, axis=-1, keepdims=True)
    comb = (w[0:1, 0:1] * l0_ref[...] + w[0:1, 1:2] * l1_ref[...]
            + w[0:1, 2:3] * cur2)

    logits = jnp.dot(comb, ow_ref[...],
                     preferred_element_type=jnp.float32) + ob_ref[...]
    mx = jnp.max(logits, axis=-1, keepdims=True)
    sh = logits - mx
    lse = jnp.log(jnp.sum(jnp.exp(sh), axis=-1, keepdims=True))
    out_ref[...] = sh - lse


def kernel(x, adj_m, input_W, input_b, conv_W, conv_b, bn_gamma, bn_beta,
           output_W, output_b, layer_weights):
    n, d_in = x.shape
    h = input_W.shape[1]
    e = adj_m.shape[1]
    out_dim = output_W.shape[1]
    nl = conv_W.shape[0]

    npad = ((n + 1 + 127) // 128) * 128
    tb = NS * (NB0 + NB1)
    tbp = tb + (NB0 - NB1)
    ep = tbp * BLK * CH

    src = adj_m[0]
    dst = adj_m[1]
    pad = jnp.full((ep - e,), n, dtype=jnp.int32)
    srcp = jnp.concatenate([src, pad]).reshape(tbp, BLK, CH)
    dstp = jnp.concatenate([dst, pad]).reshape(tbp, BLK, CH)

    deg_k = _make_deg_kernel(npad)
    scat_k = _make_scatter_kernel(npad, h)

    degp = deg_k(dstp)

    ib = input_b.reshape(1, h)
    xc, hs = pl.pallas_call(
        functools.partial(_tc_input_body, n, npad),
        out_shape=[
            jax.ShapeDtypeStruct((n, h), jnp.float32),
            jax.ShapeDtypeStruct((npad, h), jnp.float32),
        ],
    )(x, input_W, ib, conv_W[0], degp)

    lst = []
    for i in range(nl - 1):
        part = scat_k(hs, srcp, dstp)
        body = functools.partial(_tc_layer_body, n, npad, i > 0)
        args = [part, hs, degp, xc]
        if i > 0:
            args.append(lst[-1])
        args += [conv_b[i].reshape(1, h), bn_gamma[i].reshape(1, h),
                 bn_beta[i].reshape(1, h), conv_W[i + 1]]
        cur, hs = pl.pallas_call(
            lambda *refs, _b=body: _b(refs),
            out_shape=[
                jax.ShapeDtypeStruct((n, h), jnp.float32),
                jax.ShapeDtypeStruct((npad, h), jnp.float32),
            ],
        )(*args)
        lst.append(cur)

    part = scat_k(hs, srcp, dstp)
    lw = jnp.concatenate(
        [layer_weights.reshape(1, nl),
         jnp.full((1, h - nl), -1e30, jnp.float32)], axis=1)
    i = nl - 1
    out = pl.pallas_call(
        functools.partial(_tc_final_body, n),
        out_shape=jax.ShapeDtypeStruct((n, out_dim), jnp.float32),
    )(part, hs, degp, xc, lst[0], lst[1], conv_b[i].reshape(1, h),
      bn_gamma[i].reshape(1, h), bn_beta[i].reshape(1, h), lw,
      output_W, output_b.reshape(1, out_dim))
    return out


# speedup vs baseline: 2.6246x; 1.0034x over previous
import functools

import jax
import jax.numpy as jnp
from jax import lax
from jax.experimental import pallas as pl
from jax.experimental.pallas import tpu as pltpu
from jax.experimental.pallas import tpu_sc as plsc

NC = 2
NS = 16
NW = NC * NS
CH = 120
BLK = 6

NB0 = 22
NB1 = 6


def _make_deg_kernel(npad):
    rpt = npad // NS
    mesh = plsc.VectorSubcoreMesh(core_axis_name="c", subcore_axis_name="s")

    @functools.partial(
        pl.kernel,
        mesh=mesh,
        out_type=jax.ShapeDtypeStruct((NC, npad, 16), jnp.float32),
        scratch_types=[
            pltpu.VMEM((1, BLK, CH), jnp.int32),
            pltpu.VMEM((CH, 16), jnp.float32),
            pltpu.VMEM((CH, 16), jnp.float32),
            pltpu.VMEM_SHARED((npad, 16), jnp.float32),
        ],
    )
    def deg_kernel(dst_hbm, out_hbm, idx_d, ones_v, zero_v, acc):
        c = lax.axis_index("c")
        s = lax.axis_index("s")

        def init_rows(i, _):
            ones_v[i] = jnp.ones((16,), jnp.float32)
            zero_v[i] = jnp.zeros((16,), jnp.float32)
            return 0

        lax.fori_loop(0, CH, init_rows, 0)
        r0 = s * rpt
        nfull = rpt // CH
        rem = rpt - nfull * CH

        def zero_acc(j, _):
            pltpu.sync_copy(zero_v, acc.at[pl.ds(r0 + j * CH, CH)])
            return 0

        lax.fori_loop(0, nfull, zero_acc, 0)
        if rem:
            pltpu.sync_copy(zero_v.at[pl.ds(0, rem)],
                            acc.at[pl.ds(r0 + nfull * CH, rem)])
        plsc.subcore_barrier()

        def run(base, nb_static):
            def body(jb, _):
                pltpu.sync_copy(dst_hbm.at[base + jb], idx_d.at[0])
                for t in range(BLK):
                    pltpu.sync_copy(ones_v, acc.at[idx_d.at[0, t]], add=True)
                return 0

            lax.fori_loop(0, nb_static, body, 0)

        pl.when(c == 0)(lambda: run(s * NB0, NB0))
        pl.when(c == 1)(lambda: run(NS * NB0 + s * NB1, NB1))
        plsc.subcore_barrier()
        pltpu.sync_copy(acc.at[pl.ds(r0, rpt)], out_hbm.at[c, pl.ds(r0, rpt)])

    return deg_kernel


def _make_scatter_kernel(npad, h):
    rpt = npad // NS
    mesh = plsc.VectorSubcoreMesh(core_axis_name="c", subcore_axis_name="s")

    @functools.partial(
        pl.kernel,
        mesh=mesh,
        out_type=jax.ShapeDtypeStruct((NC, npad, h), jnp.float32),
        scratch_types=[
            pltpu.VMEM((2, BLK, CH), jnp.int32),
            pltpu.VMEM((2, BLK, CH), jnp.int32),
            pltpu.VMEM((CH, h), jnp.float32),
            pltpu.VMEM((CH, h), jnp.float32),
            pltpu.VMEM((CH, h), jnp.float32),
            pltpu.VMEM_SHARED((npad, h), jnp.float32),
            pltpu.SemaphoreType.DMA,
            pltpu.SemaphoreType.DMA,
            pltpu.SemaphoreType.DMA,
            pltpu.SemaphoreType.DMA,
            pltpu.SemaphoreType.DMA,
            pltpu.SemaphoreType.DMA,
            pltpu.SemaphoreType.DMA,
        ],
    )
    def scatter_kernel(hs_hbm, src_hbm, dst_hbm, out_hbm,
                       is2, id2, r0b, r1b, r2b, acc,
                       sg0, sg1, sg2, ss0, ss1, ss2, si):
        rows = (r0b, r1b, r2b)
        sg = (sg0, sg1, sg2)
        ss = (ss0, ss1, ss2)
        c = lax.axis_index("c")
        s = lax.axis_index("s")

        def zr(i, _):
            def zc(j, _2):
                r0b[i, pl.ds(j * 16, 16)] = jnp.zeros((16,), jnp.float32)
                return 0

            return lax.fori_loop(0, h // 16, zc, 0)

        lax.fori_loop(0, CH, zr, 0)
        r0 = s * rpt
        nfull = rpt // CH
        rem = rpt - nfull * CH

        def zero_acc(j, _):
            pltpu.sync_copy(r0b, acc.at[pl.ds(r0 + j * CH, CH)])
            return 0

        lax.fori_loop(0, nfull, zero_acc, 0)
        if rem:
            pltpu.sync_copy(r0b.at[pl.ds(0, rem)],
                            acc.at[pl.ds(r0 + nfull * CH, rem)])
        plsc.subcore_barrier()

        def process_block(bidx, slab, first_pred, tail_pred):
            oslab = 1 - slab
            for t in range(BLK):
                slot = t % 3
                nslot = (t + 2) % 3
                pltpu.make_async_copy(
                    hs_hbm.at[is2.at[slab, t]], rows[slot], sg[slot]).wait()

                def wait_prev():
                    pltpu.make_async_copy(
                        rows[nslot], acc.at[id2.at[slab, t]],
                        ss[nslot]).wait()

                if t == 0 and first_pred is not None:
                    pl.when(first_pred)(wait_prev)
                else:
                    wait_prev()
                pltpu.async_copy(rows[slot], acc.at[id2.at[slab, t]],
                                 ss[slot], add=True)

                if t == 0:
                    def prefetch():
                        pltpu.async_copy(src_hbm.at[bidx + 1],
                                         is2.at[oslab], si)
                        pltpu.async_copy(dst_hbm.at[bidx + 1],
                                         id2.at[oslab], si)

                    if tail_pred is None:
                        prefetch()
                    else:
                        pl.when(tail_pred)(prefetch)

                if t < 4:
                    pltpu.async_copy(hs_hbm.at[is2.at[slab, t + 2]],
                                     rows[nslot], sg[nslot])
                else:
                    def tail():
                        if t == 4:
                            pltpu.make_async_copy(
                                src_hbm.at[0], is2.at[oslab], si).wait()
                            pltpu.make_async_copy(
                                dst_hbm.at[0], id2.at[oslab], si).wait()
                        pltpu.async_copy(hs_hbm.at[is2.at[oslab, t - 4]],
                                         rows[nslot], sg[nslot])

                    if tail_pred is None:
                        tail()
                    else:
                        pl.when(tail_pred)(tail)

        def pipeline(base, nb_static):
            nb2s = nb_static // 2
            pltpu.sync_copy(src_hbm.at[base], is2.at[0])
            pltpu.sync_copy(dst_hbm.at[base], id2.at[0])
            pltpu.async_copy(hs_hbm.at[is2.at[0, 0]], rows[0], sg[0])
            pltpu.async_copy(hs_hbm.at[is2.at[0, 1]], rows[1], sg[1])

            def body(j2, _):
                b0 = base + 2 * j2
                process_block(b0, 0, first_pred=j2 > 0, tail_pred=None)
                process_block(b0 + 1, 1, first_pred=None,
                              tail_pred=j2 < nb2s - 1)
                return 0

            lax.fori_loop(0, nb2s, body, 0)
            pltpu.make_async_copy(rows[2], acc.at[id2.at[0, 0]],
                                  ss[2]).wait()

        pl.when(c == 0)(lambda: pipeline(s * NB0, NB0))
        pl.when(c == 1)(lambda: pipeline(NS * NB0 + s * NB1, NB1))
        plsc.subcore_barrier()
        pltpu.sync_copy(acc.at[pl.ds(r0, rpt)], out_hbm.at[c, pl.ds(r0, rpt)])

    return scatter_kernel


def _dinv_from_parts(degp, n):
    deg = degp[0, :n, 0:1] + degp[1, :n, 0:1] + 1.0
    return lax.rsqrt(deg)


def _tc_input_body(n, npad, x_ref, iw_ref, ib_ref, cw0_ref, degp_ref,
                   xc_ref, hs0_ref):
    x = x_ref[...]
    h = jnp.dot(x, iw_ref[...], preferred_element_type=jnp.float32) + ib_ref[...]
    xc_ref[...] = h
    dinv = _dinv_from_parts(degp_ref[...], n)
    hs0 = dinv * jnp.dot(h, cw0_ref[...], preferred_element_type=jnp.float32)
    hs0_ref[...] = jnp.concatenate(
        [hs0, jnp.zeros((npad - n, hs0.shape[1]), jnp.float32)], axis=0)


def _tc_layer_body(n, npad, has_prev, refs):
    if has_prev:
        (part_ref, hs_ref, degp_ref, xc_ref, prev_ref, cb_ref, g_ref, b_ref,
         cwn_ref, cur_ref, hsn_ref) = refs
    else:
        (part_ref, hs_ref, degp_ref, xc_ref, cb_ref, g_ref, b_ref,
         cwn_ref, cur_ref, hsn_ref) = refs
    part = part_ref[...]
    hs = hs_ref[...]
    dinv = _dinv_from_parts(degp_ref[...], n)
    agg = dinv * (part[0, :n] + part[1, :n] + hs[:n]) + cb_ref[...]
    mu = jnp.mean(agg, axis=0, keepdims=True)
    ce = agg - mu
    var = jnp.mean(ce * ce, axis=0, keepdims=True)
    bn = g_ref[...] * ce * lax.rsqrt(var + 1e-5) + b_ref[...]
    r = jnp.maximum(bn, 0.0)
    cur = r + 0.2 * xc_ref[...]
    if has_prev:
        cur = cur + 0.5 * prev_ref[...]
    cur_ref[...] = cur
    hsn = dinv * jnp.dot(cur, cwn_ref[...], preferred_element_type=jnp.float32)
    hsn_ref[...] = jnp.concatenate(
        [hsn, jnp.zeros((npad - n, hsn.shape[1]), jnp.float32)], axis=0)


def _tc_final_body(n, part_ref, hs_ref, degp_ref, xc_ref, l0_ref, l1_ref,
                   cb_ref, g_ref, b_ref, lw_ref, ow_ref, ob_ref, out_ref):
    part = part_ref[...]
    hs = hs_ref[...]
    dinv = _dinv_from_parts(degp_ref[...], n)
    agg = dinv * (part[0, :n] + part[1, :n] + hs[:n]) + cb_ref[...]
    mu = jnp.mean(agg, axis=0, keepdims=True)
    ce = agg - mu
    var = jnp.mean(ce * ce, axis=0, keepdims=True)
    bn = g_ref[...] * ce * lax.rsqrt(var + 1e-5) + b_ref[...]
    r = jnp.maximum(bn, 0.0)
    cur2 = r + 0.2 * xc_ref[...] + 0.5 * l1_ref[...]

    lw = lw_ref[...]
    m = jnp.max(lw, axis=-1, keepdims=True)
    e = jnp.exp(lw - m)
    w = e / jnp.sum(e, axis=-1, keepdims=True)
    comb = (w[0:1, 0:1] * l0_ref[...] + w[0:1, 1:2] * l1_ref[...]
            + w[0:1, 2:3] * cur2)

    logits = jnp.dot(comb, ow_ref[...],
                     preferred_element_type=jnp.float32) + ob_ref[...]
    mx = jnp.max(logits, axis=-1, keepdims=True)
    sh = logits - mx
    lse = jnp.log(jnp.sum(jnp.exp(sh), axis=-1, keepdims=True))
    out_ref[...] = sh - lse


def kernel(x, adj_m, input_W, input_b, conv_W, conv_b, bn_gamma, bn_beta,
           output_W, output_b, layer_weights):
    n, d_in = x.shape
    h = input_W.shape[1]
    e = adj_m.shape[1]
    out_dim = output_W.shape[1]
    nl = conv_W.shape[0]

    npad = ((n + 1 + 127) // 128) * 128
    tb = NS * (NB0 + NB1)
    tbp = tb + (NB0 - NB1)
    ep = tbp * BLK * CH

    src = adj_m[0]
    dst = adj_m[1]
    pad = jnp.full((ep - e,), n, dtype=jnp.int32)
    srcp = jnp.concatenate([src, pad]).reshape(tbp, BLK, CH)
    dstp = jnp.concatenate([dst, pad]).reshape(tbp, BLK, CH)

    deg_k = _make_deg_kernel(npad)
    scat_k = _make_scatter_kernel(npad, h)

    degp = deg_k(dstp)

    ib = input_b.reshape(1, h)
    xc, hs = pl.pallas_call(
        functools.partial(_tc_input_body, n, npad),
        out_shape=[
            jax.ShapeDtypeStruct((n, h), jnp.float32),
            jax.ShapeDtypeStruct((npad, h), jnp.float32),
        ],
    )(x, input_W, ib, conv_W[0], degp)

    lst = []
    for i in range(nl - 1):
        part = scat_k(hs, srcp, dstp)
        body = functools.partial(_tc_layer_body, n, npad, i > 0)
        args = [part, hs, degp, xc]
        if i > 0:
            args.append(lst[-1])
        args += [conv_b[i].reshape(1, h), bn_gamma[i].reshape(1, h),
                 bn_beta[i].reshape(1, h), conv_W[i + 1]]
        cur, hs = pl.pallas_call(
            lambda *refs, _b=body: _b(refs),
            out_shape=[
                jax.ShapeDtypeStruct((n, h), jnp.float32),
                jax.ShapeDtypeStruct((npad, h), jnp.float32),
            ],
        )(*args)
        lst.append(cur)

    part = scat_k(hs, srcp, dstp)
    lw = jnp.concatenate(
        [layer_weights.reshape(1, nl),
         jnp.full((1, h - nl), -1e30, jnp.float32)], axis=1)
    i = nl - 1
    out = pl.pallas_call(
        functools.partial(_tc_final_body, n),
        out_shape=jax.ShapeDtypeStruct((n, out_dim), jnp.float32),
    )(part, hs, degp, xc, lst[0], lst[1], conv_b[i].reshape(1, h),
      bn_gamma[i].reshape(1, h), bn_beta[i].reshape(1, h), lw,
      output_W, output_b.reshape(1, out_dim))
    return out
